# Initial kernel scaffold; baseline (speedup 1.0000x reference)
import jax
import jax.numpy as jnp
from jax.experimental import pallas as pl


def kernel(x, W1, b1, W2, b2):
    raise NotImplementedError("write your pallas kernel here")


# speedup vs baseline: 36.8400x; 36.8400x over previous
import functools

import jax
import jax.numpy as jnp
from jax.experimental import pallas as pl
from jax.experimental.pallas import tpu as pltpu

_B, _N, _D, _K = 16, 576, 384, 5
_NEG = -3e38


def _body(x_ref, w1_ref, b1_ref, w2_ref, b2_ref, out_ref):
    x = x_ref[0]
    norm = jnp.sqrt(jnp.sum(x * x, axis=1, keepdims=True)) + 1e-8
    xn = x / norm
    S = jax.lax.dot_general(xn, xn, (((1,), (1,)), ((), ())),
                            preferred_element_type=jnp.float32)
    row = jax.lax.broadcasted_iota(jnp.int32, (_N, _N), 0)
    col = jax.lax.broadcasted_iota(jnp.int32, (_N, _N), 1)
    S = jnp.where(row == col, _NEG, S)

    A = jnp.zeros((_N, _N), jnp.float32)
    denom = jnp.zeros((_N, 1), jnp.float32)
    v0 = None
    for i in range(_K):
        m = jnp.max(S, axis=1, keepdims=True)
        if i == 0:
            v0 = m
        jmin = jnp.min(jnp.where(S == m, col, _N), axis=1, keepdims=True)
        onehot = col == jmin
        w = jnp.exp(m - v0)
        A = A + jnp.where(onehot, w, 0.0)
        denom = denom + w
        S = jnp.where(onehot, _NEG, S)
    A = A / denom

    agg = jax.lax.dot_general(A, x, (((1,), (0,)), ((), ())),
                              preferred_element_type=jnp.float32)
    h = x + agg
    h1 = jax.lax.dot_general(h, w1_ref[...], (((1,), (1,)), ((), ())),
                             preferred_element_type=jnp.float32)
    h1 = jnp.maximum(h1 + b1_ref[...], 0.0)
    h2 = jax.lax.dot_general(h1, w2_ref[...], (((1,), (1,)), ((), ())),
                             preferred_element_type=jnp.float32)
    out_ref[0] = jnp.maximum(h2 + b2_ref[...], 0.0)


@jax.jit
def kernel(x, W1, b1, W2, b2):
    B, N, D = x.shape
    H = W1.shape[0]
    b1r = b1.reshape(1, H)
    b2r = b2.reshape(1, H)
    return pl.pallas_call(
        _body,
        grid=(B,),
        in_specs=[
            pl.BlockSpec((1, N, D), lambda b: (b, 0, 0)),
            pl.BlockSpec((H, D), lambda b: (0, 0)),
            pl.BlockSpec((1, H), lambda b: (0, 0)),
            pl.BlockSpec((H, H), lambda b: (0, 0)),
            pl.BlockSpec((1, H), lambda b: (0, 0)),
        ],
        out_specs=pl.BlockSpec((1, N, H), lambda b: (b, 0, 0)),
        out_shape=jax.ShapeDtypeStruct((B, N, H), jnp.float32),
        compiler_params=pltpu.CompilerParams(
            dimension_semantics=("arbitrary",),
        ),
    )(x, W1, b1r, W2, b2r)
